# Initial kernel scaffold; baseline (speedup 1.0000x reference)
import jax
import jax.numpy as jnp
from jax.experimental import pallas as pl


def kernel(batch_x, last_edge_index, cau_data):
    raise NotImplementedError("write your pallas kernel here")


# speedup vs baseline: 1.3215x; 1.3215x over previous
import functools

import jax
import jax.numpy as jnp
from jax import lax
from jax.experimental import pallas as pl
from jax.experimental.pallas import tpu as pltpu
from jax.experimental.pallas import tpu_sc as plsc

K_EDGES = 30
N_NODES = 256
N_BATCH = 8
N_LAG = 96
T_LEN = 2000

SC_CORES = 2
SC_SUBCORES = 16
SC_LANES = 16
N_WORKERS = SC_CORES * SC_SUBCORES
ROWS_PER_W = N_NODES // N_WORKERS
VREGS_PER_ROW = N_NODES // SC_LANES

_I32_MIN = -2147483648


def _prep_body(cau_ref, bx_ref, cos_ref, bxT_ref):
    c = cau_ref[...]
    cmin = jnp.min(c, axis=0, keepdims=True)
    cmax = jnp.max(c, axis=0, keepdims=True)
    cau = (c - cmin) / (cmax - cmin + 1e-8)
    ct = jnp.transpose(cau, (1, 0))
    g = jnp.dot(ct, cau, preferred_element_type=jnp.float32)
    nsq = jnp.sum(ct * ct, axis=1, keepdims=True)
    norms_r = jnp.sqrt(nsq)
    norms_c = jnp.transpose(norms_r, (1, 0))
    cos = g / (norms_r * norms_c + 1e-8)
    row_i = lax.broadcasted_iota(jnp.int32, (N_NODES, N_NODES), 0)
    col_i = lax.broadcasted_iota(jnp.int32, (N_NODES, N_NODES), 1)
    cos_ref[...] = cos - jnp.where(row_i == col_i, 1.0, 0.0).astype(jnp.float32)
    for b in range(N_BATCH):
        bxT_ref[b] = jnp.transpose(bx_ref[b], (1, 0))


def _prep(cau_data, batch_x):
    return pl.pallas_call(
        _prep_body,
        out_shape=(
            jax.ShapeDtypeStruct((N_NODES, N_NODES), jnp.float32),
            jax.ShapeDtypeStruct((N_BATCH, N_NODES, N_LAG), jnp.float32),
        ),
    )(cau_data, batch_x)


def _f32_sortable_key(v):
    s = lax.bitcast_convert_type(v, jnp.int32)
    return jnp.where(s < 0, jnp.int32(_I32_MIN) - s, s)


def _topk_row_adj(keys):
    mn = keys[0]
    mx = keys[0]
    for i in range(1, VREGS_PER_ROW):
        mn = jnp.minimum(mn, keys[i])
        mx = jnp.maximum(mx, keys[i])
    lo0 = jnp.min(mn)
    hi0 = jnp.max(mx)

    def count_ge(t):
        tv = jnp.full((SC_LANES,), t, dtype=jnp.int32)
        acc = jnp.zeros((SC_LANES,), dtype=jnp.int32)
        for i in range(VREGS_PER_ROW):
            acc = acc + jnp.where(keys[i] >= tv, 1, 0).astype(jnp.int32)
        return jnp.sum(acc)

    def body(_, carry):
        lo, hi = carry
        mid = lo + (hi - lo + 1) // 2
        cnt = count_ge(mid)
        big = cnt >= K_EDGES
        return jnp.where(big, mid, lo), jnp.where(big, hi, mid - 1)

    lo, hi = lax.fori_loop(0, 32, body, (lo0, hi0))
    thr = lo
    thr_v = jnp.full((SC_LANES,), thr, dtype=jnp.int32)

    n_gt = count_ge(thr + 1)
    need = K_EDGES - n_gt
    out = []
    base = jnp.int32(0)
    for i in range(VREGS_PER_ROW):
        gt = keys[i] > thr_v
        eq = keys[i] == thr_v
        eq_i = jnp.where(eq, 1, 0).astype(jnp.int32)
        pref = plsc.cumsum(eq_i)
        take = eq & ((base + pref) <= need)
        base = base + jnp.sum(eq_i)
        out.append(jnp.where(gt | take, 1.0, 0.0).astype(jnp.float32))
    return out


def _topk_body(cos_hbm, adj_hbm, rows_v, adj_v, sem):
    wid = lax.axis_index("s") * SC_CORES + lax.axis_index("c")
    row0 = wid * ROWS_PER_W
    pltpu.sync_copy(cos_hbm.at[pl.ds(row0, ROWS_PER_W)], rows_v)
    for r in range(ROWS_PER_W):
        keys = [
            _f32_sortable_key(rows_v[r, pl.ds(i * SC_LANES, SC_LANES)])
            for i in range(VREGS_PER_ROW)
        ]
        adj_row = _topk_row_adj(keys)
        for i in range(VREGS_PER_ROW):
            adj_v[r, pl.ds(i * SC_LANES, SC_LANES)] = adj_row[i]
    pltpu.sync_copy(adj_v, adj_hbm.at[pl.ds(row0, ROWS_PER_W)])


def _topk_adj(cos):
    mesh = plsc.VectorSubcoreMesh(
        core_axis_name="c", subcore_axis_name="s",
        num_cores=SC_CORES, num_subcores=SC_SUBCORES)
    krn = pl.kernel(
        _topk_body,
        out_type=jax.ShapeDtypeStruct((N_NODES, N_NODES), jnp.float32),
        mesh=mesh,
        compiler_params=pltpu.CompilerParams(needs_layout_passes=False),
        scratch_types=[
            pltpu.VMEM((ROWS_PER_W, N_NODES), jnp.float32),
            pltpu.VMEM((ROWS_PER_W, N_NODES), jnp.float32),
            pltpu.SemaphoreType.DMA,
        ],
    )
    return krn(cos)


def _lap_kron_body(adj_ref, bl_ref, sl_ref, l_ref):
    i = pl.program_id(0)
    j = pl.program_id(1)

    @pl.when((i == 0) & (j == 0))
    def _():
        a = adj_ref[...]
        a = jnp.maximum(a, jnp.transpose(a, (1, 0)))
        row_i = lax.broadcasted_iota(jnp.int32, (N_NODES, N_NODES), 0)
        col_i = lax.broadcasted_iota(jnp.int32, (N_NODES, N_NODES), 1)
        eye = jnp.where(row_i == col_i, 1.0, 0.0).astype(jnp.float32)
        sl = jnp.maximum(a, eye)
        sl_ref[...] = sl
        deg = jnp.sum(sl, axis=1, keepdims=True)
        dinv_r = jnp.where(deg > 0, lax.rsqrt(deg), 0.0)
        dinv_c = jnp.transpose(dinv_r, (1, 0))
        l_ref[...] = eye - dinv_r * sl * dinv_c

    sel = jnp.where(i == j, 1.0, 0.0).astype(jnp.float32)
    bl_ref[...] = l_ref[...] * sel


def _lap_kron(adj):
    grid = (N_BATCH, N_BATCH)
    return pl.pallas_call(
        _lap_kron_body,
        grid=grid,
        in_specs=[pl.BlockSpec((N_NODES, N_NODES), lambda i, j: (0, 0))],
        out_specs=(
            pl.BlockSpec((N_NODES, N_NODES), lambda i, j: (i, j)),
            pl.BlockSpec((N_NODES, N_NODES), lambda i, j: (0, 0)),
            pl.BlockSpec((N_NODES, N_NODES), lambda i, j: (0, 0)),
        ),
        out_shape=(
            jax.ShapeDtypeStruct((N_BATCH * N_NODES, N_BATCH * N_NODES),
                                 jnp.float32),
            jax.ShapeDtypeStruct((N_NODES, N_NODES), jnp.float32),
            jax.ShapeDtypeStruct((N_NODES, N_NODES), jnp.float32),
        ),
    )(adj)


@jax.jit
def kernel(batch_x, last_edge_index, cau_data):
    del last_edge_index
    cos, bxT = _prep(cau_data, batch_x)
    adj = _topk_adj(cos)
    batch_l, selfloop_adj, l_sym = _lap_kron(adj)
    pyg_x = bxT.reshape(N_BATCH * N_NODES, N_LAG)
    return (pyg_x, selfloop_adj, batch_l, l_sym, bxT)


# speedup vs baseline: 1.4698x; 1.1122x over previous
import functools

import jax
import jax.numpy as jnp
from jax import lax
from jax.experimental import pallas as pl
from jax.experimental.pallas import tpu as pltpu
from jax.experimental.pallas import tpu_sc as plsc

K_EDGES = 30
N_NODES = 256
N_BATCH = 8
N_LAG = 96
T_LEN = 2000

SC_CORES = 2
SC_SUBCORES = 16
SC_LANES = 16
N_WORKERS = SC_CORES * SC_SUBCORES
ROWS_PER_W = N_NODES // N_WORKERS
VREGS_PER_ROW = N_NODES // SC_LANES

_I32_MIN = -2147483648


def _prep_body(cau_ref, bx_ref, cos_ref, bxT_ref):
    c = cau_ref[...]
    cmin = jnp.min(c, axis=0, keepdims=True)
    cmax = jnp.max(c, axis=0, keepdims=True)
    cau = (c - cmin) / (cmax - cmin + 1e-8)
    ct = jnp.transpose(cau, (1, 0))
    g = jnp.dot(ct, cau, preferred_element_type=jnp.float32)
    nsq = jnp.sum(ct * ct, axis=1, keepdims=True)
    norms_r = jnp.sqrt(nsq)
    norms_c = jnp.transpose(norms_r, (1, 0))
    cos = g / (norms_r * norms_c + 1e-8)
    row_i = lax.broadcasted_iota(jnp.int32, (N_NODES, N_NODES), 0)
    col_i = lax.broadcasted_iota(jnp.int32, (N_NODES, N_NODES), 1)
    cos_ref[...] = cos - jnp.where(row_i == col_i, 1.0, 0.0).astype(jnp.float32)
    for b in range(N_BATCH):
        bxT_ref[b] = jnp.transpose(bx_ref[b], (1, 0))


def _prep(cau_data, batch_x):
    return pl.pallas_call(
        _prep_body,
        out_shape=(
            jax.ShapeDtypeStruct((N_NODES, N_NODES), jnp.float32),
            jax.ShapeDtypeStruct((N_BATCH, N_NODES, N_LAG), jnp.float32),
        ),
    )(cau_data, batch_x)


def _f32_sortable_key(v):
    s = lax.bitcast_convert_type(v, jnp.int32)
    return jnp.where(s < 0, jnp.int32(_I32_MIN) - s, s)


_KEY_BOUND = 1069547520


def _topk_row_adj(keys):
    def count_ge(tv):
        acc = plsc.all_reduce_population_count(keys[0] >= tv)
        for i in range(1, VREGS_PER_ROW):
            acc = acc + plsc.all_reduce_population_count(keys[i] >= tv)
        return acc

    k_splat = jnp.full((SC_LANES,), K_EDGES, dtype=jnp.int32)
    lo0 = jnp.full((SC_LANES,), -_KEY_BOUND, dtype=jnp.int32)
    hi0 = jnp.full((SC_LANES,), _KEY_BOUND, dtype=jnp.int32)

    def body(_, carry):
        lo, hi = carry
        mid = lo + lax.shift_right_arithmetic(hi - lo + 1, 1)
        big = count_ge(mid) >= k_splat
        return (jnp.where(big, mid, lo), jnp.where(big, hi, mid - 1))

    thr_v, _ = lax.fori_loop(0, 31, body, (lo0, hi0))

    need = k_splat - count_ge(thr_v + 1)
    out = []
    base = jnp.zeros((SC_LANES,), dtype=jnp.int32)
    for i in range(VREGS_PER_ROW):
        gt = keys[i] > thr_v
        eq = keys[i] == thr_v
        pref = plsc.cumsum(jnp.where(eq, 1, 0).astype(jnp.int32))
        take = eq & ((base + pref) <= need)
        base = base + plsc.all_reduce_population_count(eq)
        out.append(jnp.where(gt | take, 1.0, 0.0).astype(jnp.float32))
    return out


def _topk_body(cos_hbm, adj_hbm, rows_v, adj_v, sem):
    wid = lax.axis_index("s") * SC_CORES + lax.axis_index("c")
    row0 = wid * ROWS_PER_W
    pltpu.sync_copy(cos_hbm.at[pl.ds(row0, ROWS_PER_W)], rows_v)
    for r in range(ROWS_PER_W):
        keys = [
            _f32_sortable_key(rows_v[r, pl.ds(i * SC_LANES, SC_LANES)])
            for i in range(VREGS_PER_ROW)
        ]
        adj_row = _topk_row_adj(keys)
        for i in range(VREGS_PER_ROW):
            adj_v[r, pl.ds(i * SC_LANES, SC_LANES)] = adj_row[i]
    pltpu.sync_copy(adj_v, adj_hbm.at[pl.ds(row0, ROWS_PER_W)])


def _topk_adj(cos):
    mesh = plsc.VectorSubcoreMesh(
        core_axis_name="c", subcore_axis_name="s",
        num_cores=SC_CORES, num_subcores=SC_SUBCORES)
    krn = pl.kernel(
        _topk_body,
        out_type=jax.ShapeDtypeStruct((N_NODES, N_NODES), jnp.float32),
        mesh=mesh,
        compiler_params=pltpu.CompilerParams(needs_layout_passes=False),
        scratch_types=[
            pltpu.VMEM((ROWS_PER_W, N_NODES), jnp.float32),
            pltpu.VMEM((ROWS_PER_W, N_NODES), jnp.float32),
            pltpu.SemaphoreType.DMA,
        ],
    )
    return krn(cos)


def _lap_kron_body(adj_ref, bl_ref, sl_ref, l_ref):
    i = pl.program_id(0)
    j = pl.program_id(1)

    @pl.when((i == 0) & (j == 0))
    def _():
        a = adj_ref[...]
        a = jnp.maximum(a, jnp.transpose(a, (1, 0)))
        row_i = lax.broadcasted_iota(jnp.int32, (N_NODES, N_NODES), 0)
        col_i = lax.broadcasted_iota(jnp.int32, (N_NODES, N_NODES), 1)
        eye = jnp.where(row_i == col_i, 1.0, 0.0).astype(jnp.float32)
        sl = jnp.maximum(a, eye)
        sl_ref[...] = sl
        deg = jnp.sum(sl, axis=1, keepdims=True)
        dinv_r = jnp.where(deg > 0, lax.rsqrt(deg), 0.0)
        dinv_c = jnp.transpose(dinv_r, (1, 0))
        l_ref[...] = eye - dinv_r * sl * dinv_c

    sel = jnp.where(i == j, 1.0, 0.0).astype(jnp.float32)
    bl_ref[...] = l_ref[...] * sel


def _lap_kron(adj):
    grid = (N_BATCH, N_BATCH)
    return pl.pallas_call(
        _lap_kron_body,
        grid=grid,
        in_specs=[pl.BlockSpec((N_NODES, N_NODES), lambda i, j: (0, 0))],
        out_specs=(
            pl.BlockSpec((N_NODES, N_NODES), lambda i, j: (i, j)),
            pl.BlockSpec((N_NODES, N_NODES), lambda i, j: (0, 0)),
            pl.BlockSpec((N_NODES, N_NODES), lambda i, j: (0, 0)),
        ),
        out_shape=(
            jax.ShapeDtypeStruct((N_BATCH * N_NODES, N_BATCH * N_NODES),
                                 jnp.float32),
            jax.ShapeDtypeStruct((N_NODES, N_NODES), jnp.float32),
            jax.ShapeDtypeStruct((N_NODES, N_NODES), jnp.float32),
        ),
    )(adj)


@jax.jit
def kernel(batch_x, last_edge_index, cau_data):
    del last_edge_index
    cos, bxT = _prep(cau_data, batch_x)
    adj = _topk_adj(cos)
    batch_l, selfloop_adj, l_sym = _lap_kron(adj)
    pyg_x = bxT.reshape(N_BATCH * N_NODES, N_LAG)
    return (pyg_x, selfloop_adj, batch_l, l_sym, bxT)


# speedup vs baseline: 2.1174x; 1.4406x over previous
import functools

import jax
import jax.numpy as jnp
from jax import lax
from jax.experimental import pallas as pl
from jax.experimental.pallas import tpu as pltpu
from jax.experimental.pallas import tpu_sc as plsc

K_EDGES = 30
N_NODES = 256
N_BATCH = 8
N_LAG = 96
T_LEN = 2000

SC_CORES = 2
SC_SUBCORES = 16
SC_LANES = 16
N_WORKERS = SC_CORES * SC_SUBCORES
ROWS_PER_W = N_NODES // N_WORKERS
VREGS_PER_ROW = N_NODES // SC_LANES

_I32_MIN = -2147483648


def _prep_body(cau_ref, cos_ref):
    c = cau_ref[...]
    cmin = jnp.min(c, axis=0, keepdims=True)
    cmax = jnp.max(c, axis=0, keepdims=True)
    cau = (c - cmin) / (cmax - cmin + 1e-8)
    ct = jnp.transpose(cau, (1, 0))
    g = jnp.dot(ct, cau, preferred_element_type=jnp.float32)
    nsq = jnp.sum(ct * ct, axis=1, keepdims=True)
    norms_r = jnp.sqrt(nsq)
    norms_c = jnp.transpose(norms_r, (1, 0))
    cos = g / (norms_r * norms_c + 1e-8)
    row_i = lax.broadcasted_iota(jnp.int32, (N_NODES, N_NODES), 0)
    col_i = lax.broadcasted_iota(jnp.int32, (N_NODES, N_NODES), 1)
    cos_ref[...] = cos - jnp.where(row_i == col_i, 1.0, 0.0).astype(jnp.float32)


def _prep(cau_data):
    return pl.pallas_call(
        _prep_body,
        out_shape=jax.ShapeDtypeStruct((N_NODES, N_NODES), jnp.float32),
    )(cau_data)


def _fill_body(bx_ref, blz_ref, bxT_ref):
    i = pl.program_id(0)
    blz_ref[...] = jnp.zeros((N_NODES, N_BATCH * N_NODES), jnp.float32)

    @pl.when(i == 0)
    def _():
        for b in range(N_BATCH):
            bxT_ref[b] = jnp.transpose(bx_ref[b], (1, 0))


def _fill(batch_x):
    return pl.pallas_call(
        _fill_body,
        grid=(N_BATCH,),
        in_specs=[pl.BlockSpec((N_BATCH, N_LAG, N_NODES), lambda i: (0, 0, 0))],
        out_specs=(
            pl.BlockSpec((N_NODES, N_BATCH * N_NODES), lambda i: (i, 0)),
            pl.BlockSpec((N_BATCH, N_NODES, N_LAG), lambda i: (0, 0, 0)),
        ),
        out_shape=(
            jax.ShapeDtypeStruct((N_BATCH * N_NODES, N_BATCH * N_NODES),
                                 jnp.float32),
            jax.ShapeDtypeStruct((N_BATCH, N_NODES, N_LAG), jnp.float32),
        ),
    )(batch_x)


def _f32_sortable_key(v):
    s = lax.bitcast_convert_type(v, jnp.int32)
    return jnp.where(s < 0, jnp.int32(_I32_MIN) - s, s)


_KEY_BOUND = 1069547520


def _topk_row_adj(keys):
    def count_ge(tv):
        acc = plsc.all_reduce_population_count(keys[0] >= tv)
        for i in range(1, VREGS_PER_ROW):
            acc = acc + plsc.all_reduce_population_count(keys[i] >= tv)
        return acc

    k_splat = jnp.full((SC_LANES,), K_EDGES, dtype=jnp.int32)
    lo0 = jnp.full((SC_LANES,), -_KEY_BOUND, dtype=jnp.int32)
    hi0 = jnp.full((SC_LANES,), _KEY_BOUND, dtype=jnp.int32)

    def body(_, carry):
        lo, hi = carry
        mid = lo + lax.shift_right_arithmetic(hi - lo + 1, 1)
        big = count_ge(mid) >= k_splat
        return (jnp.where(big, mid, lo), jnp.where(big, hi, mid - 1))

    thr_v, _ = lax.fori_loop(0, 31, body, (lo0, hi0))

    need = k_splat - count_ge(thr_v + 1)
    out = []
    base = jnp.zeros((SC_LANES,), dtype=jnp.int32)
    for i in range(VREGS_PER_ROW):
        gt = keys[i] > thr_v
        eq = keys[i] == thr_v
        pref = plsc.cumsum(jnp.where(eq, 1, 0).astype(jnp.int32))
        take = eq & ((base + pref) <= need)
        base = base + plsc.all_reduce_population_count(eq)
        out.append(jnp.where(gt | take, 1.0, 0.0).astype(jnp.float32))
    return out


def _topk_body(cos_hbm, adj_hbm, rows_v, adj_v, sem):
    wid = lax.axis_index("s") * SC_CORES + lax.axis_index("c")
    row0 = wid * ROWS_PER_W
    pltpu.sync_copy(cos_hbm.at[pl.ds(row0, ROWS_PER_W)], rows_v)
    for r in range(ROWS_PER_W):
        keys = [
            _f32_sortable_key(rows_v[r, pl.ds(i * SC_LANES, SC_LANES)])
            for i in range(VREGS_PER_ROW)
        ]
        adj_row = _topk_row_adj(keys)
        for i in range(VREGS_PER_ROW):
            adj_v[r, pl.ds(i * SC_LANES, SC_LANES)] = adj_row[i]
    pltpu.sync_copy(adj_v, adj_hbm.at[pl.ds(row0, ROWS_PER_W)])


def _topk_adj(cos):
    mesh = plsc.VectorSubcoreMesh(
        core_axis_name="c", subcore_axis_name="s",
        num_cores=SC_CORES, num_subcores=SC_SUBCORES)
    krn = pl.kernel(
        _topk_body,
        out_type=jax.ShapeDtypeStruct((N_NODES, N_NODES), jnp.float32),
        mesh=mesh,
        compiler_params=pltpu.CompilerParams(needs_layout_passes=False),
        scratch_types=[
            pltpu.VMEM((ROWS_PER_W, N_NODES), jnp.float32),
            pltpu.VMEM((ROWS_PER_W, N_NODES), jnp.float32),
            pltpu.SemaphoreType.DMA,
        ],
    )
    return krn(cos)


def _lap_diag_body(adj_ref, blz_ref, bl_ref, sl_ref, l_ref):
    del blz_ref
    i = pl.program_id(0)

    @pl.when(i == 0)
    def _():
        a = adj_ref[...]
        a = jnp.maximum(a, jnp.transpose(a, (1, 0)))
        row_i = lax.broadcasted_iota(jnp.int32, (N_NODES, N_NODES), 0)
        col_i = lax.broadcasted_iota(jnp.int32, (N_NODES, N_NODES), 1)
        eye = jnp.where(row_i == col_i, 1.0, 0.0).astype(jnp.float32)
        sl = jnp.maximum(a, eye)
        sl_ref[...] = sl
        deg = jnp.sum(sl, axis=1, keepdims=True)
        dinv_r = jnp.where(deg > 0, lax.rsqrt(deg), 0.0)
        dinv_c = jnp.transpose(dinv_r, (1, 0))
        l_ref[...] = eye - dinv_r * sl * dinv_c

    bl_ref[...] = l_ref[...]


def _lap_diag(adj, bl_zeros):
    return pl.pallas_call(
        _lap_diag_body,
        grid=(N_BATCH,),
        in_specs=[
            pl.BlockSpec((N_NODES, N_NODES), lambda i: (0, 0)),
            pl.BlockSpec((N_NODES, N_NODES), lambda i: (0, 0)),
        ],
        out_specs=(
            pl.BlockSpec((N_NODES, N_NODES), lambda i: (i, i)),
            pl.BlockSpec((N_NODES, N_NODES), lambda i: (0, 0)),
            pl.BlockSpec((N_NODES, N_NODES), lambda i: (0, 0)),
        ),
        out_shape=(
            jax.ShapeDtypeStruct((N_BATCH * N_NODES, N_BATCH * N_NODES),
                                 jnp.float32),
            jax.ShapeDtypeStruct((N_NODES, N_NODES), jnp.float32),
            jax.ShapeDtypeStruct((N_NODES, N_NODES), jnp.float32),
        ),
        input_output_aliases={1: 0},
    )(adj, bl_zeros)


@jax.jit
def kernel(batch_x, last_edge_index, cau_data):
    del last_edge_index
    cos = _prep(cau_data)
    adj = _topk_adj(cos)
    bl_zeros, bxT = _fill(batch_x)
    batch_l, selfloop_adj, l_sym = _lap_diag(adj, bl_zeros)
    pyg_x = bxT.reshape(N_BATCH * N_NODES, N_LAG)
    return (pyg_x, selfloop_adj, batch_l, l_sym, bxT)
